# baseline (speedup 1.0000x reference)
import functools

import jax
import jax.numpy as jnp
from jax import lax
from jax.experimental import pallas as pl
from jax.experimental.pallas import tpu as pltpu
from jax.experimental.pallas import tpu_sc as plsc

NC = 2
NS = 16
NW = NC * NS

NP = 10240
ROWS_PER_TILE = NP // NS
CHUNK = 80
DEG_LANES = 16

_HIGH = lax.Precision.HIGHEST


def _deg_body(cpt, dst2_h, ones_h, zeros_h, out_h, dstv2, onesv, zbuf, acc,
              sem):
    c = lax.axis_index("c")
    s = lax.axis_index("s")
    wid = s * NC + c
    off = s * ROWS_PER_TILE
    pltpu.sync_copy(dst2_h.at[pl.ds(wid * cpt, cpt)], dstv2)
    pltpu.sync_copy(zeros_h, zbuf)
    pltpu.sync_copy(ones_h, onesv)
    pltpu.sync_copy(zbuf, acc.at[pl.ds(off, ROWS_PER_TILE)])
    plsc.subcore_barrier()

    def fire(j, carry):
        pltpu.async_copy(onesv, acc.at[dstv2.at[j]], sem, add=True)
        return carry

    lax.fori_loop(0, cpt, fire, 0)

    def drain(j, carry):
        pltpu.make_async_copy(onesv, acc.at[dstv2.at[0]], sem).wait()
        return carry

    lax.fori_loop(0, cpt, drain, 0)
    plsc.subcore_barrier()
    pltpu.sync_copy(acc.at[pl.ds(off, ROWS_PER_TILE)], zbuf)
    pltpu.sync_copy(zbuf, out_h.at[c, pl.ds(off, ROWS_PER_TILE)])


def _make_deg_kernel(cpt):
    mesh = plsc.VectorSubcoreMesh(core_axis_name="c", subcore_axis_name="s",
                                  num_cores=NC, num_subcores=NS)
    return pl.kernel(
        functools.partial(_deg_body, cpt),
        out_type=jax.ShapeDtypeStruct((NC, NP), jnp.float32),
        mesh=mesh,
        scratch_types=[
            pltpu.VMEM((cpt, CHUNK), jnp.int32),
            pltpu.VMEM((CHUNK,), jnp.float32),
            pltpu.VMEM((ROWS_PER_TILE,), jnp.float32),
            pltpu.VMEM_SHARED((NP,), jnp.float32),
            pltpu.SemaphoreType.DMA,
        ],
    )


def _layer_body(nchunks_base, nchunks_extra, g_h, src_h, dst_h, zeros_h,
                out_h, srcv0, dstv0, srcv1, dstv1, srcv2, dstv2,
                rows0, rows1, rows2, acc, sem0, sem1, sem2):
    c = lax.axis_index("c")
    s = lax.axis_index("s")
    wid = s * NC + c
    rowbase = s * ROWS_PER_TILE
    srcv = [srcv0, srcv1, srcv2]
    dstv = [dstv0, dstv1, dstv2]
    rows = [rows0, rows1, rows2]
    sem = [sem0, sem1, sem2]
    pltpu.sync_copy(zeros_h, rows0)
    for k in range(ROWS_PER_TILE // CHUNK):
        pltpu.sync_copy(rows0, acc.at[pl.ds(rowbase + k * CHUNK, CHUNK)])
    nch = nchunks_base + (wid < nchunks_extra).astype(jnp.int32)

    def idx_load(j, b):
        off = (wid + NW * j) * CHUNK
        pltpu.sync_copy(src_h.at[pl.ds(off, CHUNK)], srcv[b])
        pltpu.sync_copy(dst_h.at[pl.ds(off, CHUNK)], dstv[b])

    def gather_start(b):
        pltpu.async_copy(g_h.at[srcv[b]], rows[b], sem[b])

    def gather_wait(b):
        pltpu.make_async_copy(g_h.at[srcv[b]], rows[b], sem[b]).wait()

    def scatter_sync(b):
        pltpu.sync_copy(rows[b], acc.at[dstv[b]], add=True)

    for b in range(3):
        idx_load(b, b)
        gather_start(b)
    plsc.subcore_barrier()

    def trip(i, carry):
        j = 3 * i
        for p in range(3):
            jj = j + p
            gather_wait(p)
            scatter_sync(p)

            @pl.when(jj + 3 < nch)
            def _():
                idx_load(jj + 3, p)
                gather_start(p)
        return carry

    ntrips = nch // 3
    lax.fori_loop(0, ntrips, trip, 0)

    rem = nch - 3 * ntrips

    @pl.when(rem >= 1)
    def _():
        gather_wait(0)
        scatter_sync(0)

    @pl.when(rem >= 2)
    def _():
        gather_wait(1)
        scatter_sync(1)

    plsc.subcore_barrier()
    for k in range(ROWS_PER_TILE // CHUNK):
        pltpu.sync_copy(acc.at[pl.ds(rowbase + k * CHUNK, CHUNK)], rows0)
        pltpu.sync_copy(rows0, out_h.at[c, pl.ds(rowbase + k * CHUNK, CHUNK)])


def _make_layer_kernel(E, D):
    n_chunks = E // CHUNK
    assert n_chunks * CHUNK == E
    mesh = plsc.VectorSubcoreMesh(core_axis_name="c", subcore_axis_name="s",
                                  num_cores=NC, num_subcores=NS)
    return pl.kernel(
        functools.partial(_layer_body, n_chunks // NW, n_chunks % NW),
        out_type=jax.ShapeDtypeStruct((NC, NP, D), jnp.float32),
        mesh=mesh,
        scratch_types=[
            pltpu.VMEM((CHUNK,), jnp.int32),
            pltpu.VMEM((CHUNK,), jnp.int32),
            pltpu.VMEM((CHUNK,), jnp.int32),
            pltpu.VMEM((CHUNK,), jnp.int32),
            pltpu.VMEM((CHUNK,), jnp.int32),
            pltpu.VMEM((CHUNK,), jnp.int32),
            pltpu.VMEM((CHUNK, D), jnp.float32),
            pltpu.VMEM((CHUNK, D), jnp.float32),
            pltpu.VMEM((CHUNK, D), jnp.float32),
            pltpu.VMEM_SHARED((NP, D), jnp.float32),
            pltpu.SemaphoreType.DMA,
            pltpu.SemaphoreType.DMA,
            pltpu.SemaphoreType.DMA,
        ],
    )


def _inv_from_degp(degp_blk):
    deg = degp_blk[0] + degp_blk[1]
    return jnp.where(deg > 0, 1.0 / jnp.sqrt(jnp.maximum(deg, 1.0)), 0.0)


def _tc_in_body(x_ref, w_ref, b_ref, degp_ref, g0_ref):
    inv = _inv_from_degp(degp_ref[...])
    h = jnp.tanh(
        jnp.dot(x_ref[...], w_ref[...], preferred_element_type=jnp.float32) + b_ref[...])
    g0_ref[...] = h * inv[:, None]


def _tc_mid_body(p_ref, degp_ref, w_ref, b_ref, g_ref):
    inv = _inv_from_degp(degp_ref[...])
    agg = (p_ref[0] + p_ref[1]) * inv[:, None]
    h = jnp.tanh(
        jnp.dot(agg, w_ref[...], preferred_element_type=jnp.float32) + b_ref[...])
    g_ref[...] = h * inv[:, None]


def _tc_out_body(p_ref, degp_ref, wg_ref, bg_ref, wp_ref, bp_ref, wv_ref,
                 pi_ref, vf_ref):
    i = pl.program_id(0)
    inv = _inv_from_degp(degp_ref[...])
    agg = (p_ref[0] + p_ref[1]) * inv[:, None]
    h = jnp.tanh(
        jnp.dot(agg, wg_ref[...], preferred_element_type=jnp.float32) + bg_ref[...])
    pi_ref[...] = jnp.dot(h, wp_ref[...], preferred_element_type=jnp.float32) + bp_ref[...]
    part = jnp.sum(h * wv_ref[...]).reshape(1, 1)

    @pl.when(i == 0)
    def _():
        vf_ref[...] = part

    @pl.when(i > 0)
    def _():
        vf_ref[...] += part


def _row_grid_specs(R, D):
    row = pl.BlockSpec((R, D), lambda i: (i, 0))
    part = pl.BlockSpec((NC, R, D), lambda i: (0, i, 0))
    degp = pl.BlockSpec((NC, R), lambda i: (0, i))
    mat = pl.BlockSpec((D, D), lambda i: (0, 0))
    vec = pl.BlockSpec((1, D), lambda i: (0, 0))
    return row, part, degp, mat, vec


def _tc_in(x_p, W, b2, degP, R=1024):
    D = x_p.shape[1]
    row, part, degp, mat, vec = _row_grid_specs(R, D)
    return pl.pallas_call(
        _tc_in_body,
        grid=(NP // R,),
        in_specs=[row, mat, vec, degp],
        out_specs=row,
        out_shape=jax.ShapeDtypeStruct((NP, D), jnp.float32),
    )(x_p, W, b2, degP)


def _tc_mid(P, degP, W, b2, R=1024):
    D = P.shape[2]
    row, part, degp, mat, vec = _row_grid_specs(R, D)
    return pl.pallas_call(
        _tc_mid_body,
        grid=(NP // R,),
        in_specs=[part, degp, mat, vec],
        out_specs=row,
        out_shape=jax.ShapeDtypeStruct((NP, D), jnp.float32),
    )(P, degP, W, b2)


def _tc_out(P, degP, Wg, bg2, Wp, bp2, Wv2, R=1024):
    D = P.shape[2]
    row, part, degp, mat, vec = _row_grid_specs(R, D)
    scal = pl.BlockSpec((1, 1), lambda i: (0, 0))
    return pl.pallas_call(
        _tc_out_body,
        grid=(NP // R,),
        in_specs=[part, degp, mat, vec, mat, vec, row],
        out_specs=[row, scal],
        out_shape=[
            jax.ShapeDtypeStruct((NP, D), jnp.float32),
            jax.ShapeDtypeStruct((1, 1), jnp.float32),
        ],
    )(P, degP, Wg, bg2, Wp, bp2, Wv2)


def kernel(x, edge_index, W_in, b_in, W_g1, b_g1, W_g2, b_g2, W_pol, b_pol,
           W_val, b_val):
    N, D = x.shape
    E = edge_index.shape[1]
    src = edge_index[0]
    dst = edge_index[1]

    cpt = (-(-E // (NW * CHUNK)) + 7) // 8 * 8
    e_pad = NW * cpt * CHUNK - E
    dst2 = jnp.concatenate([dst, jnp.full((e_pad,), NP - 1, jnp.int32)])
    dst2 = dst2.reshape(NW * cpt, CHUNK)

    pad = NP - N
    x_p = jnp.concatenate([x, jnp.zeros((pad, D), x.dtype)], axis=0)
    Wv2 = jnp.concatenate(
        [W_val.reshape(N, D), jnp.zeros((pad, D), W_val.dtype)], axis=0)

    ones_deg = jnp.ones((CHUNK,), jnp.float32)
    zeros_deg = jnp.zeros((ROWS_PER_TILE,), jnp.float32)
    zeros_row = jnp.zeros((CHUNK, D), jnp.float32)

    b_in2 = b_in.reshape(1, D)
    b_g12 = b_g1.reshape(1, D)
    b_g22 = b_g2.reshape(1, D)
    b_pol2 = b_pol.reshape(1, D)

    deg_kernel = _make_deg_kernel(cpt)
    layer_kernel = _make_layer_kernel(E, D)

    degP = deg_kernel(dst2, ones_deg, zeros_deg)
    g0 = _tc_in(x_p, W_in, b_in2, degP)
    P1 = layer_kernel(g0, src, dst, zeros_row)
    g1 = _tc_mid(P1, degP, W_g1, b_g12)
    P2 = layer_kernel(g1, src, dst, zeros_row)
    pi_p, vf = _tc_out(P2, degP, W_g2, b_g22, W_pol, b_pol2, Wv2)

    return pi_p[:N], vf[0, 0] + b_val
